# baseline (speedup 1.0000x reference)
import jax
import jax.numpy as jnp
from jax.experimental import pallas as pl
from jax.experimental.pallas import tpu as pltpu

N = 256
IN_DIM = 128
HIDDEN = 128
OUT_DIM = 128
HEADS = 4
BATCH = 4
BJ = 16

_PREC = jax.lax.Precision.HIGHEST
_SPREC = jax.lax.Precision.DEFAULT


def _gat_layer(xin, maskf, Wl, bl, Wr, br, att_wide, bias):
    D = att_wide.shape[1]
    xl = jnp.dot(xin, Wl, precision=_PREC) + bl
    xr = jnp.dot(xin, Wr, precision=_PREC) + br
    u = xl + xr
    u = jnp.maximum(u, 0.2 * u)
    sdiag = jnp.dot(u, att_wide, precision=_PREC)

    kk = jax.lax.broadcasted_iota(jnp.int32, (BJ, BJ * N), 1)
    jj = jax.lax.broadcasted_iota(jnp.int32, (BJ, BJ * N), 0)
    same_j = (kk // N) == jj

    outs = []
    for j0 in range(0, N, BJ):
        xr_blk = xr[j0:j0 + BJ]
        t = xr_blk[:, None, :] + xl[None, :, :]
        t = jnp.maximum(t, 0.2 * t)
        s = jnp.dot(t.reshape(BJ * N, D), att_wide, precision=_SPREC)
        ex = jnp.exp(s.reshape(BJ, N, D) - sdiag[j0:j0 + BJ][:, None, :])
        p = ex * xl[None, :, :]
        lm = jnp.where(same_j, jnp.tile(maskf[j0:j0 + BJ], (1, BJ)), 0.0)
        denom = jnp.dot(lm, ex.reshape(BJ * N, D), precision=_SPREC)
        numer = jnp.dot(lm, p.reshape(BJ * N, D), precision=_SPREC)
        outs.append(numer / (denom + 1e-16))
    return jnp.concatenate(outs, axis=0) + bias


def _fused_kernel(x_ref, emb_ref,
                  wl1_ref, bl1_ref, wr1_ref, br1_ref, aw1_ref, bias1_ref,
                  wl2_ref, bl2_ref, wr2_ref, br2_ref, aw2_ref, bias2_ref,
                  out_ref):
    x = x_ref[0]
    emb = emb_ref[...]

    nrm = jnp.maximum(
        jnp.sqrt(jnp.sum(emb * emb, axis=1, keepdims=True)), 1e-12)
    ne = emb / nrm
    adj = jnp.dot(ne, ne.T, precision=_PREC)
    rows = jax.lax.broadcasted_iota(jnp.int32, (N, N), 0)
    cols = jax.lax.broadcasted_iota(jnp.int32, (N, N), 1)
    maskf = jnp.where((adj != 0.0) | (rows == cols), 1.0, 0.0)

    h = _gat_layer(x, maskf, wl1_ref[...], bl1_ref[...],
                   wr1_ref[...], br1_ref[...], aw1_ref[...], bias1_ref[...])
    h = jnp.maximum(h, 0.0)
    o = _gat_layer(h, maskf, wl2_ref[...], bl2_ref[...],
                   wr2_ref[...], br2_ref[...], aw2_ref[...], bias2_ref[...])
    out_ref[0] = o


def _widen_att(att):
    H, hd = att.shape
    eye = jnp.eye(H, dtype=att.dtype)
    w = jnp.einsum('hd,hg,e->hdge', att, eye, jnp.ones((hd,), att.dtype))
    return w.reshape(H * hd, H * hd)


@jax.jit
def kernel(x, embedding, W_l1, b_l1, W_r1, b_r1, att1, bias1,
           W_l2, b_l2, W_r2, b_r2, att2, bias2):
    aw1 = _widen_att(att1)
    aw2 = _widen_att(att2)
    b2 = lambda a: a.reshape(1, -1)

    full = lambda s: pl.BlockSpec(s, lambda b: (0,) * len(s))
    grid_spec = pl.GridSpec(
        grid=(BATCH,),
        in_specs=[
            pl.BlockSpec((1, N, IN_DIM), lambda b: (b, 0, 0)),
            full((N, HIDDEN)),
            full((IN_DIM, HIDDEN)), full((1, HIDDEN)),
            full((IN_DIM, HIDDEN)), full((1, HIDDEN)),
            full((HIDDEN, HIDDEN)), full((1, HIDDEN)),
            full((HIDDEN, OUT_DIM)), full((1, OUT_DIM)),
            full((HIDDEN, OUT_DIM)), full((1, OUT_DIM)),
            full((OUT_DIM, OUT_DIM)), full((1, OUT_DIM)),
        ],
        out_specs=pl.BlockSpec((1, N, OUT_DIM), lambda b: (b, 0, 0)),
    )
    return pl.pallas_call(
        _fused_kernel,
        grid_spec=grid_spec,
        out_shape=jax.ShapeDtypeStruct((BATCH, N, OUT_DIM), jnp.float32),
        compiler_params=pltpu.CompilerParams(
            dimension_semantics=("parallel",)),
    )(x, embedding,
      W_l1, b2(b_l1), W_r1, b2(b_r1), aw1, b2(bias1),
      W_l2, b2(b_l2), W_r2, b2(b_r2), aw2, b2(bias2))


# speedup vs baseline: 1.0922x; 1.0922x over previous
import jax
import jax.numpy as jnp
from jax.experimental import pallas as pl
from jax.experimental.pallas import tpu as pltpu

N = 256
IN_DIM = 128
HIDDEN = 128
OUT_DIM = 128
HEADS = 4
BATCH = 4
BJ = 32

_PREC = jax.lax.Precision.HIGHEST
_SPREC = jax.lax.Precision.DEFAULT


def _gat_layer(xin, lm_scr, Wl, bl, Wr, br, att_wide, bias):
    D = att_wide.shape[1]
    xl = jnp.dot(xin, Wl, precision=_PREC) + bl
    xr = jnp.dot(xin, Wr, precision=_PREC) + br
    u = xl + xr
    u = jnp.maximum(u, 0.2 * u)
    sdiag = jnp.dot(u, att_wide, precision=_PREC)

    xl16 = xl.astype(jnp.bfloat16)
    xr16 = xr.astype(jnp.bfloat16)
    aw16 = att_wide.astype(jnp.bfloat16)
    outs = []
    for j0 in range(0, N, BJ):
        xr_blk = xr16[j0:j0 + BJ]
        t = xr_blk[:, None, :] + xl16[None, :, :]
        t = jnp.maximum(t, jnp.bfloat16(0.2) * t)
        s = jnp.dot(t.reshape(BJ * N, D), aw16,
                    preferred_element_type=jnp.float32)
        ex = jnp.exp(s.reshape(BJ, N, D) - sdiag[j0:j0 + BJ][:, None, :])
        p = ex * xl[None, :, :]
        lm = lm_scr[j0:j0 + BJ]
        denom = jnp.dot(lm, ex.reshape(BJ * N, D), precision=_SPREC)
        numer = jnp.dot(lm, p.reshape(BJ * N, D), precision=_SPREC)
        outs.append(numer / (denom + 1e-16))
    return jnp.concatenate(outs, axis=0) + bias


def _fused_kernel(x_ref, emb_ref,
                  wl1_ref, bl1_ref, wr1_ref, br1_ref, aw1_ref, bias1_ref,
                  wl2_ref, bl2_ref, wr2_ref, br2_ref, aw2_ref, bias2_ref,
                  out_ref, lm_scr):
    x = x_ref[0]

    @pl.when(pl.program_id(0) == 0)
    def _build_selection():
        emb = emb_ref[...]
        nrm = jnp.maximum(
            jnp.sqrt(jnp.sum(emb * emb, axis=1, keepdims=True)), 1e-12)
        ne = emb / nrm
        adj = jnp.dot(ne, ne.T, precision=_PREC)
        rows = jax.lax.broadcasted_iota(jnp.int32, (N, N), 0)
        cols = jax.lax.broadcasted_iota(jnp.int32, (N, N), 1)
        maskf = jnp.where((adj != 0.0) | (rows == cols), 1.0, 0.0)
        kk = jax.lax.broadcasted_iota(jnp.int32, (BJ, BJ * N), 1)
        jj = jax.lax.broadcasted_iota(jnp.int32, (BJ, BJ * N), 0)
        same_j = (kk // N) == jj
        for j0 in range(0, N, BJ):
            lm_scr[j0:j0 + BJ] = jnp.where(
                same_j, jnp.tile(maskf[j0:j0 + BJ], (1, BJ)), 0.0)

    h = _gat_layer(x, lm_scr, wl1_ref[...], bl1_ref[...],
                   wr1_ref[...], br1_ref[...], aw1_ref[...], bias1_ref[...])
    h = jnp.maximum(h, 0.0)
    o = _gat_layer(h, lm_scr, wl2_ref[...], bl2_ref[...],
                   wr2_ref[...], br2_ref[...], aw2_ref[...], bias2_ref[...])
    out_ref[0] = o


def _widen_att(att):
    H, hd = att.shape
    eye = jnp.eye(H, dtype=att.dtype)
    w = jnp.einsum('hd,hg,e->hdge', att, eye, jnp.ones((hd,), att.dtype))
    return w.reshape(H * hd, H * hd)


@jax.jit
def kernel(x, embedding, W_l1, b_l1, W_r1, b_r1, att1, bias1,
           W_l2, b_l2, W_r2, b_r2, att2, bias2):
    aw1 = _widen_att(att1)
    aw2 = _widen_att(att2)
    b2 = lambda a: a.reshape(1, -1)

    full = lambda s: pl.BlockSpec(s, lambda b: (0,) * len(s))
    grid_spec = pl.GridSpec(
        grid=(BATCH,),
        in_specs=[
            pl.BlockSpec((1, N, IN_DIM), lambda b: (b, 0, 0)),
            full((N, HIDDEN)),
            full((IN_DIM, HIDDEN)), full((1, HIDDEN)),
            full((IN_DIM, HIDDEN)), full((1, HIDDEN)),
            full((HIDDEN, HIDDEN)), full((1, HIDDEN)),
            full((HIDDEN, OUT_DIM)), full((1, OUT_DIM)),
            full((HIDDEN, OUT_DIM)), full((1, OUT_DIM)),
            full((OUT_DIM, OUT_DIM)), full((1, OUT_DIM)),
        ],
        out_specs=pl.BlockSpec((1, N, OUT_DIM), lambda b: (b, 0, 0)),
        scratch_shapes=[pltpu.VMEM((N, BJ * N), jnp.float32)],
    )
    return pl.pallas_call(
        _fused_kernel,
        grid_spec=grid_spec,
        out_shape=jax.ShapeDtypeStruct((BATCH, N, OUT_DIM), jnp.float32),
        compiler_params=pltpu.CompilerParams(
            dimension_semantics=("arbitrary",)),
    )(x, embedding,
      W_l1, b2(b_l1), W_r1, b2(b_r1), aw1, b2(bias1),
      W_l2, b2(b_l2), W_r2, b2(b_r2), aw2, b2(bias2))


# speedup vs baseline: 1.1079x; 1.0144x over previous
import jax
import jax.numpy as jnp
from jax.experimental import pallas as pl
from jax.experimental.pallas import tpu as pltpu

N = 256
IN_DIM = 128
HIDDEN = 128
OUT_DIM = 128
HEADS = 4
BATCH = 4
BJ = 32

_PREC = jax.lax.Precision.HIGHEST
_SPREC = jax.lax.Precision.DEFAULT


def _gat_layer(xin, lm_scr, Wl, bl, Wr, br, att_wide, bias):
    D = att_wide.shape[1]
    xl = jnp.dot(xin, Wl, precision=_PREC) + bl
    xr = jnp.dot(xin, Wr, precision=_PREC) + br
    u = xl + xr
    u = jnp.maximum(u, 0.2 * u)
    sdiag = jnp.dot(u, att_wide, precision=_PREC)

    xl16 = xl.astype(jnp.bfloat16)
    xr16 = xr.astype(jnp.bfloat16)
    aw16 = att_wide.astype(jnp.bfloat16)
    outs = []
    for j0 in range(0, N, BJ):
        xr_blk = xr16[j0:j0 + BJ]
        t = xr_blk[:, None, :] + xl16[None, :, :]
        t = jnp.maximum(t, jnp.bfloat16(0.2) * t)
        s = jnp.dot(t.reshape(BJ * N, D), aw16,
                    preferred_element_type=jnp.float32)
        ex = jnp.exp(s.reshape(BJ, N, D) - sdiag[j0:j0 + BJ][:, None, :])
        exb = ex.astype(jnp.bfloat16)
        p = exb * xl16[None, :, :]
        lm = lm_scr[j0:j0 + BJ]
        denom = jnp.dot(lm, exb.reshape(BJ * N, D),
                        preferred_element_type=jnp.float32)
        numer = jnp.dot(lm, p.reshape(BJ * N, D),
                        preferred_element_type=jnp.float32)
        outs.append(numer / (denom + 1e-16))
    return jnp.concatenate(outs, axis=0) + bias


def _fused_kernel(x_ref, emb_ref,
                  wl1_ref, bl1_ref, wr1_ref, br1_ref, aw1_ref, bias1_ref,
                  wl2_ref, bl2_ref, wr2_ref, br2_ref, aw2_ref, bias2_ref,
                  out_ref, lm_scr):
    x = x_ref[0]

    @pl.when(pl.program_id(0) == 0)
    def _build_selection():
        emb = emb_ref[...]
        nrm = jnp.maximum(
            jnp.sqrt(jnp.sum(emb * emb, axis=1, keepdims=True)), 1e-12)
        ne = emb / nrm
        adj = jnp.dot(ne, ne.T, precision=_PREC)
        rows = jax.lax.broadcasted_iota(jnp.int32, (N, N), 0)
        cols = jax.lax.broadcasted_iota(jnp.int32, (N, N), 1)
        maskf = jnp.where((adj != 0.0) | (rows == cols), 1.0, 0.0)
        kk = jax.lax.broadcasted_iota(jnp.int32, (BJ, BJ * N), 1)
        jj = jax.lax.broadcasted_iota(jnp.int32, (BJ, BJ * N), 0)
        same_j = (kk // N) == jj
        for j0 in range(0, N, BJ):
            lm_scr[j0:j0 + BJ] = jnp.where(
                same_j, jnp.tile(maskf[j0:j0 + BJ], (1, BJ)),
                0.0).astype(jnp.bfloat16)

    h = _gat_layer(x, lm_scr, wl1_ref[...], bl1_ref[...],
                   wr1_ref[...], br1_ref[...], aw1_ref[...], bias1_ref[...])
    h = jnp.maximum(h, 0.0)
    o = _gat_layer(h, lm_scr, wl2_ref[...], bl2_ref[...],
                   wr2_ref[...], br2_ref[...], aw2_ref[...], bias2_ref[...])
    out_ref[0] = o


def _widen_att(att):
    H, hd = att.shape
    eye = jnp.eye(H, dtype=att.dtype)
    w = jnp.einsum('hd,hg,e->hdge', att, eye, jnp.ones((hd,), att.dtype))
    return w.reshape(H * hd, H * hd)


@jax.jit
def kernel(x, embedding, W_l1, b_l1, W_r1, b_r1, att1, bias1,
           W_l2, b_l2, W_r2, b_r2, att2, bias2):
    aw1 = _widen_att(att1)
    aw2 = _widen_att(att2)
    b2 = lambda a: a.reshape(1, -1)

    full = lambda s: pl.BlockSpec(s, lambda b: (0,) * len(s))
    grid_spec = pl.GridSpec(
        grid=(BATCH,),
        in_specs=[
            pl.BlockSpec((1, N, IN_DIM), lambda b: (b, 0, 0)),
            full((N, HIDDEN)),
            full((IN_DIM, HIDDEN)), full((1, HIDDEN)),
            full((IN_DIM, HIDDEN)), full((1, HIDDEN)),
            full((HIDDEN, HIDDEN)), full((1, HIDDEN)),
            full((HIDDEN, OUT_DIM)), full((1, OUT_DIM)),
            full((HIDDEN, OUT_DIM)), full((1, OUT_DIM)),
            full((OUT_DIM, OUT_DIM)), full((1, OUT_DIM)),
        ],
        out_specs=pl.BlockSpec((1, N, OUT_DIM), lambda b: (b, 0, 0)),
        scratch_shapes=[pltpu.VMEM((N, BJ * N), jnp.bfloat16)],
    )
    return pl.pallas_call(
        _fused_kernel,
        grid_spec=grid_spec,
        out_shape=jax.ShapeDtypeStruct((BATCH, N, OUT_DIM), jnp.float32),
        compiler_params=pltpu.CompilerParams(
            dimension_semantics=("arbitrary",)),
    )(x, embedding,
      W_l1, b2(b_l1), W_r1, b2(b_r1), aw1, b2(bias1),
      W_l2, b2(b_l2), W_r2, b2(b_r2), aw2, b2(bias2))


# speedup vs baseline: 1.2037x; 1.0865x over previous
import jax
import jax.numpy as jnp
from jax.experimental import pallas as pl
from jax.experimental.pallas import tpu as pltpu

N = 256
IN_DIM = 128
HIDDEN = 128
OUT_DIM = 128
HEADS = 4
BATCH = 4
BJ = 32

_PREC = jax.lax.Precision.HIGHEST
_SPREC = jax.lax.Precision.DEFAULT


def _gat_layer(xin, lm_scr, Wl, bl, Wr, br, att_wide, bias):
    D = att_wide.shape[1]
    xl = jnp.dot(xin, Wl, precision=_PREC) + bl
    xr = jnp.dot(xin, Wr, precision=_PREC) + br
    u = xl + xr
    u = jnp.maximum(u, 0.2 * u)
    u16 = u.astype(jnp.bfloat16)

    xl16 = xl.astype(jnp.bfloat16)
    xr16 = xr.astype(jnp.bfloat16)
    aw16 = att_wide.astype(jnp.bfloat16)
    outs = []
    for j0 in range(0, N, BJ):
        xr_blk = xr16[j0:j0 + BJ]
        t = xr_blk[:, None, :] + xl16[None, :, :]
        t = jnp.maximum(t, jnp.bfloat16(0.2) * t)
        t = t - u16[j0:j0 + BJ][:, None, :]
        s = jnp.dot(t.reshape(BJ * N, D), aw16,
                    preferred_element_type=jnp.float32)
        ex = jnp.exp(s.reshape(BJ, N, D))
        exb = ex.astype(jnp.bfloat16)
        p = exb * xl16[None, :, :]
        lm = lm_scr[j0:j0 + BJ]
        denom = jnp.dot(lm, exb.reshape(BJ * N, D),
                        preferred_element_type=jnp.float32)
        numer = jnp.dot(lm, p.reshape(BJ * N, D),
                        preferred_element_type=jnp.float32)
        outs.append(numer / (denom + 1e-16))
    return jnp.concatenate(outs, axis=0) + bias


def _fused_kernel(x_ref, emb_ref,
                  wl1_ref, bl1_ref, wr1_ref, br1_ref, aw1_ref, bias1_ref,
                  wl2_ref, bl2_ref, wr2_ref, br2_ref, aw2_ref, bias2_ref,
                  out_ref, lm_scr):
    x = x_ref[0]

    @pl.when(pl.program_id(0) == 0)
    def _build_selection():
        emb = emb_ref[...]
        nrm = jnp.maximum(
            jnp.sqrt(jnp.sum(emb * emb, axis=1, keepdims=True)), 1e-12)
        ne = emb / nrm
        adj = jnp.dot(ne, ne.T, precision=_PREC)
        rows = jax.lax.broadcasted_iota(jnp.int32, (N, N), 0)
        cols = jax.lax.broadcasted_iota(jnp.int32, (N, N), 1)
        maskf = jnp.where((adj != 0.0) | (rows == cols), 1.0, 0.0)
        kk = jax.lax.broadcasted_iota(jnp.int32, (BJ, BJ * N), 1)
        jj = jax.lax.broadcasted_iota(jnp.int32, (BJ, BJ * N), 0)
        same_j = (kk // N) == jj
        for j0 in range(0, N, BJ):
            lm_scr[j0:j0 + BJ] = jnp.where(
                same_j, jnp.tile(maskf[j0:j0 + BJ], (1, BJ)),
                0.0).astype(jnp.bfloat16)

    h = _gat_layer(x, lm_scr, wl1_ref[...], bl1_ref[...],
                   wr1_ref[...], br1_ref[...], aw1_ref[...], bias1_ref[...])
    h = jnp.maximum(h, 0.0)
    o = _gat_layer(h, lm_scr, wl2_ref[...], bl2_ref[...],
                   wr2_ref[...], br2_ref[...], aw2_ref[...], bias2_ref[...])
    out_ref[0] = o


def _widen_att(att):
    H, hd = att.shape
    eye = jnp.eye(H, dtype=att.dtype)
    w = jnp.einsum('hd,hg,e->hdge', att, eye, jnp.ones((hd,), att.dtype))
    return w.reshape(H * hd, H * hd)


@jax.jit
def kernel(x, embedding, W_l1, b_l1, W_r1, b_r1, att1, bias1,
           W_l2, b_l2, W_r2, b_r2, att2, bias2):
    aw1 = _widen_att(att1)
    aw2 = _widen_att(att2)
    b2 = lambda a: a.reshape(1, -1)

    full = lambda s: pl.BlockSpec(s, lambda b: (0,) * len(s))
    grid_spec = pl.GridSpec(
        grid=(BATCH,),
        in_specs=[
            pl.BlockSpec((1, N, IN_DIM), lambda b: (b, 0, 0)),
            full((N, HIDDEN)),
            full((IN_DIM, HIDDEN)), full((1, HIDDEN)),
            full((IN_DIM, HIDDEN)), full((1, HIDDEN)),
            full((HIDDEN, HIDDEN)), full((1, HIDDEN)),
            full((HIDDEN, OUT_DIM)), full((1, OUT_DIM)),
            full((HIDDEN, OUT_DIM)), full((1, OUT_DIM)),
            full((OUT_DIM, OUT_DIM)), full((1, OUT_DIM)),
        ],
        out_specs=pl.BlockSpec((1, N, OUT_DIM), lambda b: (b, 0, 0)),
        scratch_shapes=[pltpu.VMEM((N, BJ * N), jnp.bfloat16)],
    )
    return pl.pallas_call(
        _fused_kernel,
        grid_spec=grid_spec,
        out_shape=jax.ShapeDtypeStruct((BATCH, N, OUT_DIM), jnp.float32),
        compiler_params=pltpu.CompilerParams(
            dimension_semantics=("arbitrary",)),
    )(x, embedding,
      W_l1, b2(b_l1), W_r1, b2(b_r1), aw1, b2(bias1),
      W_l2, b2(b_l2), W_r2, b2(b_r2), aw2, b2(bias2))
